# baseline (speedup 1.0000x reference)
import functools

import jax
import jax.numpy as jnp
from jax import lax
from jax.experimental import pallas as pl
from jax.experimental.pallas import tpu as pltpu
from jax.experimental.pallas import tpu_sc as plsc

N = 10000
D = 128
H = 16
E = 320000

NC = 2
NS = 16
CHUNK = 128
CPW0 = 128
CPW1 = 32
CPWMAX = max(CPW0, CPW1)
TOTC = NS * (CPW0 + CPW1)
EP = TOTC * CHUNK
NBUF = 8
GA = NBUF // 2

NP = 10240
RPS = NP // NS
PR = NP * H // 128


def _seg_body(compute_cnt, src_hbm, dst_hbm, y_hbm, *rest):
    if compute_cnt:
        part_out, cnt_out, src_v, dst_v, rows_v, ones_v, zrow_v, \
            acc_sh, cnt_sh, gsem, ssem, csem = rest
    else:
        part_out, src_v, dst_v, rows_v, zrow_v, acc_sh, gsem, ssem = rest

    c = lax.axis_index("c")
    s = lax.axis_index("s")
    base = lax.select(c == 0, s * CPW0, NS * CPW0 + s * CPW1)

    idesc = [
        pltpu.async_copy(src_hbm.at[pl.ds(base, CPWMAX)], src_v, gsem.at[0]),
        pltpu.async_copy(dst_hbm.at[pl.ds(base, CPWMAX)], dst_v, gsem.at[1]),
    ]

    def fill_const(i, _):
        zrow_v[i] = jnp.zeros((16,), jnp.float32)
        if compute_cnt:
            ones_v[i] = jnp.ones((16,), jnp.float32)
        return 0
    lax.fori_loop(0, CHUNK, fill_const, 0)

    for k in range(RPS // CHUNK):
        pltpu.async_copy(zrow_v, acc_sh.at[pl.ds(s * RPS + k * CHUNK, CHUNK)],
                         ssem.at[k])
        if compute_cnt:
            pltpu.async_copy(
                zrow_v, cnt_sh.at[pl.ds(s * RPS + k * CHUNK, CHUNK)],
                csem.at[k % 8])
    for k in range(RPS // CHUNK):
        pltpu.make_async_copy(
            zrow_v, acc_sh.at[pl.ds(s * RPS + k * CHUNK, CHUNK)],
            ssem.at[k]).wait()
        if compute_cnt:
            pltpu.make_async_copy(
                zrow_v, cnt_sh.at[pl.ds(s * RPS + k * CHUNK, CHUNK)],
                csem.at[k % 8]).wait()
    for d in idesc:
        d.wait()
    plsc.subcore_barrier()

    def gather(j, b):
        pltpu.async_copy(y_hbm.at[src_v.at[j]], rows_v.at[b], gsem.at[b])

    def run(cpw):
        for b in range(GA):
            gather(b, b)

        def group(g, _):
            for b in range(NBUF):
                j = g * NBUF + b
                pltpu.make_async_copy(
                    y_hbm.at[src_v.at[j]], rows_v.at[b], gsem.at[b]).wait()
                pltpu.async_copy(
                    rows_v.at[b], acc_sh.at[dst_v.at[j]], ssem.at[b],
                    add=True)
                if compute_cnt:
                    cb = b % 8

                    @pl.when(j >= 8)
                    def _():
                        pltpu.make_async_copy(
                            ones_v, cnt_sh.at[dst_v.at[j - 8]],
                            csem.at[cb]).wait()
                    pltpu.async_copy(
                        ones_v, cnt_sh.at[dst_v.at[j]], csem.at[cb],
                        add=True)

                nb = (b + GA) % NBUF

                @pl.when(j + GA < cpw)
                def _():
                    @pl.when(j >= NBUF - GA)
                    def _():
                        pltpu.make_async_copy(
                            rows_v.at[nb],
                            acc_sh.at[dst_v.at[j - (NBUF - GA)]],
                            ssem.at[nb]).wait()
                    gather(j + GA, nb)
            return 0
        lax.fori_loop(0, cpw // NBUF, group, 0)

        for b in range(NBUF):
            pltpu.make_async_copy(
                rows_v.at[b], acc_sh.at[dst_v.at[cpw - NBUF + b]],
                ssem.at[b]).wait()
        if compute_cnt:
            for cb in range(8):
                pltpu.make_async_copy(
                    ones_v, cnt_sh.at[dst_v.at[cpw - 8 + cb]],
                    csem.at[cb]).wait()

    @pl.when(c == 0)
    def _():
        run(CPW0)

    @pl.when(c == 1)
    def _():
        run(CPW1)

    plsc.subcore_barrier()

    pltpu.sync_copy(acc_sh.at[pl.ds(s * RPS, RPS)],
                    part_out.at[c, pl.ds(s * RPS, RPS)])
    if compute_cnt:
        pltpu.sync_copy(cnt_sh.at[pl.ds(s * RPS, RPS)],
                        cnt_out.at[c, pl.ds(s * RPS, RPS)])


def _make_sc_segment(compute_cnt):
    mesh = plsc.VectorSubcoreMesh(core_axis_name="c", subcore_axis_name="s")
    out_type = [jax.ShapeDtypeStruct((NC, NP, H), jnp.float32)]
    scratch = [
        pltpu.VMEM((CPWMAX, CHUNK), jnp.int32),
        pltpu.VMEM((CPWMAX, CHUNK), jnp.int32),
        pltpu.VMEM((NBUF, CHUNK, H), jnp.float32),
    ]
    if compute_cnt:
        out_type.append(jax.ShapeDtypeStruct((NC, NP, H), jnp.float32))
        scratch.append(pltpu.VMEM((CHUNK, H), jnp.float32))
    scratch.append(pltpu.VMEM((CHUNK, H), jnp.float32))
    scratch.append(pltpu.VMEM_SHARED((NP, H), jnp.float32))
    if compute_cnt:
        scratch.append(pltpu.VMEM_SHARED((NP, H), jnp.float32))
    scratch.append(pltpu.SemaphoreType.DMA((NBUF,)))
    scratch.append(pltpu.SemaphoreType.DMA((NBUF,)))
    if compute_cnt:
        scratch.append(pltpu.SemaphoreType.DMA((8,)))
    return pl.kernel(
        functools.partial(_seg_body, compute_cnt),
        out_type=tuple(out_type),
        mesh=mesh,
        scratch_types=tuple(scratch),
        compiler_params=pltpu.CompilerParams(use_tc_tiling_on_sc=False),
    )


def _tc_pre(x_p, W1l, W1r):
    def body(x_ref, wl_ref, wr_ref, y_ref, xr_ref):
        xb = jnp.concatenate(
            [x_ref[...], jnp.zeros((NP - N, D), jnp.float32)], axis=0)
        x3 = xb.reshape(PR, 8, D)
        dn = (((1,), (1,)), ((), ()))
        ys, xrs = [], []
        for q in range(8):
            xq = x3[:, q, :]
            ys.append(lax.dot_general(xq, wl_ref[...], dn,
                                      preferred_element_type=jnp.float32))
            xrs.append(lax.dot_general(xq, wr_ref[...], dn,
                                       preferred_element_type=jnp.float32))
        y_ref[...] = jnp.concatenate(ys, axis=1)
        xr_ref[...] = jnp.concatenate(xrs, axis=1)
    return pl.pallas_call(
        body,
        out_shape=(jax.ShapeDtypeStruct((PR, 128), jnp.float32),
                   jax.ShapeDtypeStruct((PR, 128), jnp.float32)),
    )(x_p, W1l, W1r)


def _tc_mid(part, cnt16, xr1p, b1t, K2l, K2r):
    def body(p_ref, c_ref, xr_ref, b_ref, kl_ref, kr_ref, y2_ref, hr_ref):
        seg = p_ref[0] + p_ref[1]
        cnt = jnp.clip(c_ref[0] + c_ref[1], 1.0, None)
        h = jax.nn.sigmoid(seg / cnt + b_ref[...] + xr_ref[...])
        y2_ref[...] = jnp.dot(h, kl_ref[...],
                              preferred_element_type=jnp.float32)
        hr_ref[...] = jnp.dot(h, kr_ref[...],
                              preferred_element_type=jnp.float32)
    return pl.pallas_call(
        body,
        out_shape=(jax.ShapeDtypeStruct((PR, 128), jnp.float32),
                   jax.ShapeDtypeStruct((PR, 128), jnp.float32)),
    )(part, cnt16, xr1p, b1t, K2l, K2r)


def _tc_final(part, cnt16, hr2p, b2t, onesk):
    def body(p_ref, c_ref, hr_ref, b_ref, ok_ref, out_ref):
        seg = p_ref[0] + p_ref[1]
        cnt = jnp.clip(c_ref[0] + c_ref[1], 1.0, None)
        z = seg / cnt + b_ref[...] + hr_ref[...]
        m = jnp.max(z, axis=1, keepdims=True)
        e = jnp.exp(z - m)
        gs = jnp.dot(e, ok_ref[...], preferred_element_type=jnp.float32)
        res = z - m - jnp.log(gs)
        rp = res[:N // 8]
        cols = [rp[:, H * q:H * (q + 1)].reshape(N // 8, 1, H)
                for q in range(8)]
        out_ref[...] = jnp.concatenate(cols, axis=1).reshape(N, H)
    return pl.pallas_call(
        body,
        out_shape=jax.ShapeDtypeStruct((N, H), jnp.float32),
    )(part, cnt16, hr2p, b2t, onesk)


@jax.jit
def kernel(x, edge_index, W1l, b1, W1r, W2l, b2, W2r):
    src = edge_index[0].astype(jnp.int32)
    dst = edge_index[1].astype(jnp.int32)
    pad = EP - E
    src_p = jnp.concatenate([src, jnp.zeros((pad,), jnp.int32)])
    dst_p = jnp.concatenate([dst, jnp.full((pad,), N, jnp.int32)])
    src_p = src_p.reshape(TOTC, CHUNK)
    dst_p = dst_p.reshape(TOTC, CHUNK)

    eye8 = jnp.eye(8, dtype=jnp.float32)
    K2l = jnp.kron(eye8, W2l.T)
    K2r = jnp.kron(eye8, W2r.T)
    onesk = jnp.kron(eye8, jnp.ones((H, H), jnp.float32))
    b1t = jnp.tile(b1, 8).reshape(1, 128)
    b2t = jnp.tile(b2, 8).reshape(1, 128)

    y1p, xr1p = _tc_pre(x, W1l, W1r)
    part1, cnt16 = _make_sc_segment(True)(src_p, dst_p,
                                          y1p.reshape(NP, H))
    p1 = part1.reshape(NC, PR, 128)
    c16 = cnt16.reshape(NC, PR, 128)
    y2p, hr2p = _tc_mid(p1, c16, xr1p, b1t, K2l, K2r)
    (part2,) = _make_sc_segment(False)(src_p, dst_p, y2p.reshape(NP, H))
    return _tc_final(part2.reshape(NC, PR, 128), c16, hr2p, b2t, onesk)


# speedup vs baseline: 1.0295x; 1.0295x over previous
import functools

import jax
import jax.numpy as jnp
from jax import lax
from jax.experimental import pallas as pl
from jax.experimental.pallas import tpu as pltpu
from jax.experimental.pallas import tpu_sc as plsc

N = 10000
D = 128
H = 16
E = 320000

NC = 2
NS = 16
CHUNK = 128
CPW0 = 112
CPW1 = 48
CPWMAX = max(CPW0, CPW1)
TOTC = NS * (CPW0 + CPW1)
EP = TOTC * CHUNK
NBUF = 8
GA = NBUF // 2

NP = 10240
RPS = NP // NS
PR = NP * H // 128


def _seg_body(compute_cnt, src_hbm, dst_hbm, y_hbm, *rest):
    if compute_cnt:
        part_out, cnt_out, src_v, dst_v, rows_v, ones_v, zrow_v, \
            acc_sh, cnt_sh, gsem, ssem, csem = rest
    else:
        part_out, src_v, dst_v, rows_v, zrow_v, acc_sh, gsem, ssem = rest

    c = lax.axis_index("c")
    s = lax.axis_index("s")
    base = lax.select(c == 0, s * CPW0, NS * CPW0 + s * CPW1)

    idesc = [
        pltpu.async_copy(src_hbm.at[pl.ds(base, CPWMAX)], src_v, gsem.at[0]),
        pltpu.async_copy(dst_hbm.at[pl.ds(base, CPWMAX)], dst_v, gsem.at[1]),
    ]

    def fill_const(i, _):
        zrow_v[i] = jnp.zeros((16,), jnp.float32)
        if compute_cnt:
            ones_v[i] = jnp.ones((16,), jnp.float32)
        return 0
    lax.fori_loop(0, CHUNK, fill_const, 0)

    for k in range(RPS // CHUNK):
        pltpu.async_copy(zrow_v, acc_sh.at[pl.ds(s * RPS + k * CHUNK, CHUNK)],
                         ssem.at[k])
        if compute_cnt:
            pltpu.async_copy(
                zrow_v, cnt_sh.at[pl.ds(s * RPS + k * CHUNK, CHUNK)],
                csem.at[k % 8])
    for k in range(RPS // CHUNK):
        pltpu.make_async_copy(
            zrow_v, acc_sh.at[pl.ds(s * RPS + k * CHUNK, CHUNK)],
            ssem.at[k]).wait()
        if compute_cnt:
            pltpu.make_async_copy(
                zrow_v, cnt_sh.at[pl.ds(s * RPS + k * CHUNK, CHUNK)],
                csem.at[k % 8]).wait()
    for d in idesc:
        d.wait()
    plsc.subcore_barrier()

    def gather(j, b):
        pltpu.async_copy(y_hbm.at[src_v.at[j]], rows_v.at[b], gsem.at[b])

    def run(cpw):
        for b in range(GA):
            gather(b, b)

        def group(g, _):
            for b in range(NBUF):
                j = g * NBUF + b
                pltpu.make_async_copy(
                    y_hbm.at[src_v.at[j]], rows_v.at[b], gsem.at[b]).wait()
                pltpu.async_copy(
                    rows_v.at[b], acc_sh.at[dst_v.at[j]], ssem.at[b],
                    add=True)
                if compute_cnt:
                    cb = b % 8

                    @pl.when(j >= 8)
                    def _():
                        pltpu.make_async_copy(
                            ones_v, cnt_sh.at[dst_v.at[j - 8]],
                            csem.at[cb]).wait()
                    pltpu.async_copy(
                        ones_v, cnt_sh.at[dst_v.at[j]], csem.at[cb],
                        add=True)

                nb = (b + GA) % NBUF

                @pl.when(j + GA < cpw)
                def _():
                    @pl.when(j >= NBUF - GA)
                    def _():
                        pltpu.make_async_copy(
                            rows_v.at[nb],
                            acc_sh.at[dst_v.at[j - (NBUF - GA)]],
                            ssem.at[nb]).wait()
                    gather(j + GA, nb)
            return 0
        lax.fori_loop(0, cpw // NBUF, group, 0)

        for b in range(NBUF):
            pltpu.make_async_copy(
                rows_v.at[b], acc_sh.at[dst_v.at[cpw - NBUF + b]],
                ssem.at[b]).wait()
        if compute_cnt:
            for cb in range(8):
                pltpu.make_async_copy(
                    ones_v, cnt_sh.at[dst_v.at[cpw - 8 + cb]],
                    csem.at[cb]).wait()

    @pl.when(c == 0)
    def _():
        run(CPW0)

    @pl.when(c == 1)
    def _():
        run(CPW1)

    plsc.subcore_barrier()

    pltpu.sync_copy(acc_sh.at[pl.ds(s * RPS, RPS)],
                    part_out.at[c, pl.ds(s * RPS, RPS)])
    if compute_cnt:
        pltpu.sync_copy(cnt_sh.at[pl.ds(s * RPS, RPS)],
                        cnt_out.at[c, pl.ds(s * RPS, RPS)])


def _make_sc_segment(compute_cnt):
    mesh = plsc.VectorSubcoreMesh(core_axis_name="c", subcore_axis_name="s")
    out_type = [jax.ShapeDtypeStruct((NC, NP, H), jnp.float32)]
    scratch = [
        pltpu.VMEM((CPWMAX, CHUNK), jnp.int32),
        pltpu.VMEM((CPWMAX, CHUNK), jnp.int32),
        pltpu.VMEM((NBUF, CHUNK, H), jnp.float32),
    ]
    if compute_cnt:
        out_type.append(jax.ShapeDtypeStruct((NC, NP, H), jnp.float32))
        scratch.append(pltpu.VMEM((CHUNK, H), jnp.float32))
    scratch.append(pltpu.VMEM((CHUNK, H), jnp.float32))
    scratch.append(pltpu.VMEM_SHARED((NP, H), jnp.float32))
    if compute_cnt:
        scratch.append(pltpu.VMEM_SHARED((NP, H), jnp.float32))
    scratch.append(pltpu.SemaphoreType.DMA((NBUF,)))
    scratch.append(pltpu.SemaphoreType.DMA((NBUF,)))
    if compute_cnt:
        scratch.append(pltpu.SemaphoreType.DMA((8,)))
    return pl.kernel(
        functools.partial(_seg_body, compute_cnt),
        out_type=tuple(out_type),
        mesh=mesh,
        scratch_types=tuple(scratch),
        compiler_params=pltpu.CompilerParams(use_tc_tiling_on_sc=False),
    )


def _tc_pre(x_p, W1l, W1r):
    def body(x_ref, wl_ref, wr_ref, y_ref, xr_ref):
        xb = jnp.concatenate(
            [x_ref[...], jnp.zeros((NP - N, D), jnp.float32)], axis=0)
        x3 = xb.reshape(PR, 8, D)
        dn = (((1,), (1,)), ((), ()))
        ys, xrs = [], []
        for q in range(8):
            xq = x3[:, q, :]
            ys.append(lax.dot_general(xq, wl_ref[...], dn,
                                      preferred_element_type=jnp.float32))
            xrs.append(lax.dot_general(xq, wr_ref[...], dn,
                                       preferred_element_type=jnp.float32))
        y_ref[...] = jnp.concatenate(ys, axis=1)
        xr_ref[...] = jnp.concatenate(xrs, axis=1)
    return pl.pallas_call(
        body,
        out_shape=(jax.ShapeDtypeStruct((PR, 128), jnp.float32),
                   jax.ShapeDtypeStruct((PR, 128), jnp.float32)),
    )(x_p, W1l, W1r)


def _tc_mid(part, cnt16, xr1p, b1t, K2l, K2r):
    def body(p_ref, c_ref, xr_ref, b_ref, kl_ref, kr_ref, y2_ref, hr_ref):
        seg = p_ref[0] + p_ref[1]
        cnt = jnp.clip(c_ref[0] + c_ref[1], 1.0, None)
        h = jax.nn.sigmoid(seg / cnt + b_ref[...] + xr_ref[...])
        y2_ref[...] = jnp.dot(h, kl_ref[...],
                              preferred_element_type=jnp.float32)
        hr_ref[...] = jnp.dot(h, kr_ref[...],
                              preferred_element_type=jnp.float32)
    return pl.pallas_call(
        body,
        out_shape=(jax.ShapeDtypeStruct((PR, 128), jnp.float32),
                   jax.ShapeDtypeStruct((PR, 128), jnp.float32)),
    )(part, cnt16, xr1p, b1t, K2l, K2r)


def _tc_final(part, cnt16, hr2p, b2t, onesk):
    def body(p_ref, c_ref, hr_ref, b_ref, ok_ref, out_ref):
        seg = p_ref[0] + p_ref[1]
        cnt = jnp.clip(c_ref[0] + c_ref[1], 1.0, None)
        z = seg / cnt + b_ref[...] + hr_ref[...]
        m = jnp.max(z, axis=1, keepdims=True)
        e = jnp.exp(z - m)
        gs = jnp.dot(e, ok_ref[...], preferred_element_type=jnp.float32)
        out_ref[...] = z - m - jnp.log(gs)
    return pl.pallas_call(
        body,
        out_shape=jax.ShapeDtypeStruct((PR, 128), jnp.float32),
    )(part, cnt16, hr2p, b2t, onesk)


@jax.jit
def kernel(x, edge_index, W1l, b1, W1r, W2l, b2, W2r):
    src = edge_index[0].astype(jnp.int32)
    dst = edge_index[1].astype(jnp.int32)
    pad = EP - E
    src_p = jnp.concatenate([src, jnp.zeros((pad,), jnp.int32)])
    dst_p = jnp.concatenate([dst, jnp.full((pad,), N, jnp.int32)])
    src_p = src_p.reshape(TOTC, CHUNK)
    dst_p = dst_p.reshape(TOTC, CHUNK)

    eye8 = jnp.eye(8, dtype=jnp.float32)
    K2l = jnp.kron(eye8, W2l.T)
    K2r = jnp.kron(eye8, W2r.T)
    onesk = jnp.kron(eye8, jnp.ones((H, H), jnp.float32))
    b1t = jnp.tile(b1, 8).reshape(1, 128)
    b2t = jnp.tile(b2, 8).reshape(1, 128)

    y1p, xr1p = _tc_pre(x, W1l, W1r)
    part1, cnt16 = _make_sc_segment(True)(src_p, dst_p,
                                          y1p.reshape(NP, H))
    p1 = part1.reshape(NC, PR, 128)
    c16 = cnt16.reshape(NC, PR, 128)
    y2p, hr2p = _tc_mid(p1, c16, xr1p, b1t, K2l, K2r)
    (part2,) = _make_sc_segment(False)(src_p, dst_p, y2p.reshape(NP, H))
    outp = _tc_final(part2.reshape(NC, PR, 128), c16, hr2p, b2t, onesk)
    return outp.reshape(NP, H)[:N]


# speedup vs baseline: 1.1222x; 1.0901x over previous
import functools

import jax
import jax.numpy as jnp
from jax import lax
from jax.experimental import pallas as pl
from jax.experimental.pallas import tpu as pltpu
from jax.experimental.pallas import tpu_sc as plsc

N = 10000
D = 128
H = 16
E = 320000

NC = 2
NS = 16
CHUNK = 128
CPW0 = 112
CPW1 = 48
CPWMAX = max(CPW0, CPW1)
TOTC = NS * (CPW0 + CPW1)
EP = TOTC * CHUNK
NBUF = 8
GA = NBUF // 2

NP = 10240
RPS = NP // NS
PR = NP * H // 128


def _seg_body(compute_cnt, src_hbm, dst_hbm, y_hbm, *rest):
    if compute_cnt:
        part_out, cnt_out, src_v, dst_v, rows_v, ones_v, zrow_v, \
            acc_sh, cnt_sh, gsem, ssem, csem = rest
    else:
        part_out, src_v, dst_v, rows_v, zrow_v, acc_sh, gsem, ssem = rest

    c = lax.axis_index("c")
    s = lax.axis_index("s")
    base = lax.select(c == 0, s * CPW0, NS * CPW0 + s * CPW1)

    idesc = [
        pltpu.async_copy(src_hbm.at[pl.ds(base, CPWMAX)], src_v, gsem.at[0]),
        pltpu.async_copy(dst_hbm.at[pl.ds(base, CPWMAX)], dst_v, gsem.at[1]),
    ]

    def fill_const(i, _):
        zrow_v[i] = jnp.zeros((16,), jnp.float32)
        if compute_cnt:
            ones_v[i] = jnp.ones((16,), jnp.float32)
        return 0
    lax.fori_loop(0, CHUNK, fill_const, 0)

    for k in range(RPS // CHUNK):
        pltpu.async_copy(zrow_v, acc_sh.at[pl.ds(s * RPS + k * CHUNK, CHUNK)],
                         ssem.at[k])
        if compute_cnt:
            pltpu.async_copy(
                zrow_v, cnt_sh.at[pl.ds(s * RPS + k * CHUNK, CHUNK)],
                csem.at[k % 8])
    for k in range(RPS // CHUNK):
        pltpu.make_async_copy(
            zrow_v, acc_sh.at[pl.ds(s * RPS + k * CHUNK, CHUNK)],
            ssem.at[k]).wait()
        if compute_cnt:
            pltpu.make_async_copy(
                zrow_v, cnt_sh.at[pl.ds(s * RPS + k * CHUNK, CHUNK)],
                csem.at[k % 8]).wait()
    for d in idesc:
        d.wait()
    plsc.subcore_barrier()

    def gather(j, b):
        pltpu.async_copy(y_hbm.at[src_v.at[j]], rows_v.at[b], gsem.at[b])

    def run(cpw):
        for b in range(GA):
            gather(b, b)

        def group(g, _):
            for b in range(NBUF):
                j = g * NBUF + b
                pltpu.make_async_copy(
                    y_hbm.at[src_v.at[j]], rows_v.at[b], gsem.at[b]).wait()
                pltpu.async_copy(
                    rows_v.at[b], acc_sh.at[dst_v.at[j]], ssem.at[b],
                    add=True)
                if compute_cnt:
                    cb = b % 8

                    @pl.when(j >= 8)
                    def _():
                        pltpu.make_async_copy(
                            ones_v, cnt_sh.at[dst_v.at[j - 8]],
                            csem.at[cb]).wait()
                    pltpu.async_copy(
                        ones_v, cnt_sh.at[dst_v.at[j]], csem.at[cb],
                        add=True)

                nb = (b + GA) % NBUF

                @pl.when(j + GA < cpw)
                def _():
                    @pl.when(j >= NBUF - GA)
                    def _():
                        pltpu.make_async_copy(
                            rows_v.at[nb],
                            acc_sh.at[dst_v.at[j - (NBUF - GA)]],
                            ssem.at[nb]).wait()
                    gather(j + GA, nb)
            return 0
        lax.fori_loop(0, cpw // NBUF, group, 0)

        for b in range(NBUF):
            pltpu.make_async_copy(
                rows_v.at[b], acc_sh.at[dst_v.at[cpw - NBUF + b]],
                ssem.at[b]).wait()
        if compute_cnt:
            for cb in range(8):
                pltpu.make_async_copy(
                    ones_v, cnt_sh.at[dst_v.at[cpw - 8 + cb]],
                    csem.at[cb]).wait()

    @pl.when(c == 0)
    def _():
        run(CPW0)

    @pl.when(c == 1)
    def _():
        run(CPW1)

    plsc.subcore_barrier()

    pltpu.sync_copy(acc_sh.at[pl.ds(s * RPS, RPS)],
                    part_out.at[c, pl.ds(s * RPS, RPS)])
    if compute_cnt:
        pltpu.sync_copy(cnt_sh.at[pl.ds(s * RPS, RPS)],
                        cnt_out.at[c, pl.ds(s * RPS, RPS)])


def _make_sc_segment(compute_cnt):
    mesh = plsc.VectorSubcoreMesh(core_axis_name="c", subcore_axis_name="s")
    out_type = [jax.ShapeDtypeStruct((NC, NP, H), jnp.float32)]
    scratch = [
        pltpu.VMEM((CPWMAX, CHUNK), jnp.int32),
        pltpu.VMEM((CPWMAX, CHUNK), jnp.int32),
        pltpu.VMEM((NBUF, CHUNK, H), jnp.float32),
    ]
    if compute_cnt:
        out_type.append(jax.ShapeDtypeStruct((NC, NP, H), jnp.float32))
        scratch.append(pltpu.VMEM((CHUNK, H), jnp.float32))
    scratch.append(pltpu.VMEM((CHUNK, H), jnp.float32))
    scratch.append(pltpu.VMEM_SHARED((NP, H), jnp.float32))
    if compute_cnt:
        scratch.append(pltpu.VMEM_SHARED((NP, H), jnp.float32))
    scratch.append(pltpu.SemaphoreType.DMA((NBUF,)))
    scratch.append(pltpu.SemaphoreType.DMA((NBUF,)))
    if compute_cnt:
        scratch.append(pltpu.SemaphoreType.DMA((8,)))
    return pl.kernel(
        functools.partial(_seg_body, compute_cnt),
        out_type=tuple(out_type),
        mesh=mesh,
        scratch_types=tuple(scratch),
        compiler_params=pltpu.CompilerParams(use_tc_tiling_on_sc=False),
    )


def _tc_pre(x_p, W1l, W1r):
    def body(x_ref, wl_ref, wr_ref, y_ref, xr_ref):
        x3 = x_ref[...].reshape(PR, 8, D)
        dn = (((1,), (1,)), ((), ()))
        ys, xrs = [], []
        for q in range(8):
            xq = x3[:, q, :]
            ys.append(lax.dot_general(xq, wl_ref[...], dn,
                                      preferred_element_type=jnp.float32))
            xrs.append(lax.dot_general(xq, wr_ref[...], dn,
                                       preferred_element_type=jnp.float32))
        y_ref[...] = jnp.concatenate(ys, axis=1)
        xr_ref[...] = jnp.concatenate(xrs, axis=1)
    return pl.pallas_call(
        body,
        out_shape=(jax.ShapeDtypeStruct((PR, 128), jnp.float32),
                   jax.ShapeDtypeStruct((PR, 128), jnp.float32)),
    )(x_p, W1l, W1r)


def _tc_mid(part, cnt16, xr1p, b1t, K2l, K2r):
    def body(p_ref, c_ref, xr_ref, b_ref, kl_ref, kr_ref, y2_ref, hr_ref):
        seg = p_ref[0] + p_ref[1]
        cnt = jnp.clip(c_ref[0] + c_ref[1], 1.0, None)
        h = jax.nn.sigmoid(seg / cnt + b_ref[...] + xr_ref[...])
        y2_ref[...] = jnp.dot(h, kl_ref[...],
                              preferred_element_type=jnp.float32)
        hr_ref[...] = jnp.dot(h, kr_ref[...],
                              preferred_element_type=jnp.float32)
    return pl.pallas_call(
        body,
        out_shape=(jax.ShapeDtypeStruct((PR, 128), jnp.float32),
                   jax.ShapeDtypeStruct((PR, 128), jnp.float32)),
    )(part, cnt16, xr1p, b1t, K2l, K2r)


def _tc_final(part, cnt16, hr2p, b2t, onesk):
    def body(p_ref, c_ref, hr_ref, b_ref, ok_ref, out_ref):
        seg = p_ref[0] + p_ref[1]
        cnt = jnp.clip(c_ref[0] + c_ref[1], 1.0, None)
        z = seg / cnt + b_ref[...] + hr_ref[...]
        m = jnp.max(z, axis=1, keepdims=True)
        e = jnp.exp(z - m)
        gs = jnp.dot(e, ok_ref[...], preferred_element_type=jnp.float32)
        out_ref[...] = z - m - jnp.log(gs)
    return pl.pallas_call(
        body,
        out_shape=jax.ShapeDtypeStruct((PR, 128), jnp.float32),
    )(part, cnt16, hr2p, b2t, onesk)


@jax.jit
def kernel(x, edge_index, W1l, b1, W1r, W2l, b2, W2r):
    src = edge_index[0].astype(jnp.int32)
    dst = edge_index[1].astype(jnp.int32)
    pad = EP - E
    src_p = jnp.concatenate([src, jnp.zeros((pad,), jnp.int32)])
    dst_p = jnp.concatenate([dst, jnp.full((pad,), N, jnp.int32)])
    src_p = src_p.reshape(TOTC, CHUNK)
    dst_p = dst_p.reshape(TOTC, CHUNK)
    x_p = jnp.pad(x, ((0, NP - N), (0, 0)))

    eye8 = jnp.eye(8, dtype=jnp.float32)
    K2l = jnp.kron(eye8, W2l.T)
    K2r = jnp.kron(eye8, W2r.T)
    onesk = jnp.kron(eye8, jnp.ones((H, H), jnp.float32))
    b1t = jnp.tile(b1, 8).reshape(1, 128)
    b2t = jnp.tile(b2, 8).reshape(1, 128)

    y1p, xr1p = _tc_pre(x_p, W1l, W1r)
    part1, cnt16 = _make_sc_segment(True)(src_p, dst_p,
                                          y1p.reshape(NP, H))
    p1 = part1.reshape(NC, PR, 128)
    c16 = cnt16.reshape(NC, PR, 128)
    y2p, hr2p = _tc_mid(p1, c16, xr1p, b1t, K2l, K2r)
    (part2,) = _make_sc_segment(False)(src_p, dst_p, y2p.reshape(NP, H))
    outp = _tc_final(part2.reshape(NC, PR, 128), c16, hr2p, b2t, onesk)
    return outp.reshape(NP, H)[:N]
